# Initial kernel scaffold; baseline (speedup 1.0000x reference)
import functools

import jax
import jax.numpy as jnp
from jax import lax
from jax.experimental import pallas as pl
from jax.experimental.pallas import tpu as pltpu
from jax.experimental.pallas import tpu_sc as plsc

F32 = jnp.float32

_NC = 2
_NS = 16
_NW = _NC * _NS

_CHUNK = 80


def _pre_body(x_ref, ws_ref, wd_ref, u_ref, wu_ref, b1_ref,
              xs_ref, xd_ref, c0_ref):
    x = x_ref[...]
    xs_ref[...] = jnp.dot(x, ws_ref[...], preferred_element_type=F32)
    xd_ref[...] = jnp.dot(x, wd_ref[...], preferred_element_type=F32)
    c0_ref[...] = jnp.dot(u_ref[...], wu_ref[...],
                          preferred_element_type=F32) + b1_ref[...]


def _precompute(x, ws, wd, u, wu, b1):
    n, d = x.shape
    return pl.pallas_call(
        _pre_body,
        out_shape=[
            jax.ShapeDtypeStruct((n, ws.shape[1]), F32),
            jax.ShapeDtypeStruct((n, wd.shape[1]), F32),
            jax.ShapeDtypeStruct((1, wu.shape[1]), F32),
        ],
    )(x, ws, wd, u, wu, b1)


def _make_gather(n_edges, d):
    steps = n_edges // (_NW * _CHUNK)
    epw = n_edges // _NW
    mesh = plsc.VectorSubcoreMesh(core_axis_name="c", subcore_axis_name="s")

    @functools.partial(
        pl.kernel,
        mesh=mesh,
        out_type=jax.ShapeDtypeStruct((n_edges, d), F32),
        scratch_types=[
            pltpu.VMEM((steps, _CHUNK), jnp.int32),
            pltpu.VMEM((steps, _CHUNK), jnp.int32),
            pltpu.VMEM((_CHUNK, d), F32),
            pltpu.VMEM((_CHUNK, d), F32),
            pltpu.SemaphoreType.DMA,
            pltpu.SemaphoreType.DMA,
        ],
    )
    def gather_k(xs_hbm, xd_hbm, src_hbm, dst_hbm, out_hbm,
                 sidx, didx, abuf, bbuf, sem_a, sem_b):
        wid = lax.axis_index("s") * _NC + lax.axis_index("c")
        row0 = wid * steps
        pltpu.sync_copy(src_hbm.at[pl.ds(row0, steps)], sidx)
        pltpu.sync_copy(dst_hbm.at[pl.ds(row0, steps)], didx)

        def step(j, carry):
            ca = pltpu.async_copy(xs_hbm.at[sidx.at[j]], abuf, sem_a)
            cb = pltpu.async_copy(xd_hbm.at[didx.at[j]], bbuf, sem_b)
            ca.wait()
            cb.wait()

            def add_row(r, c2):
                for cc in range(d // 16):
                    sl = pl.ds(cc * 16, 16)
                    abuf[r, sl] = abuf[r, sl] + bbuf[r, sl]
                return c2

            lax.fori_loop(0, _CHUNK, add_row, 0)
            pltpu.sync_copy(abuf,
                            out_hbm.at[pl.ds(wid * epw + j * _CHUNK, _CHUNK)])
            return carry

        lax.fori_loop(0, steps, step, 0)

    return gather_k


def _edge_body(pre_ref, ea_ref, we_ref, c0_ref, w2_ref, b2_ref,
               w3_ref, b3_ref, w4_ref, b4_ref, out_ref):
    h = (pre_ref[...]
         + jnp.dot(ea_ref[...], we_ref[...], preferred_element_type=F32)
         + c0_ref[...])
    h = jnp.maximum(h, 0.0)
    h = jnp.maximum(jnp.dot(h, w2_ref[...], preferred_element_type=F32)
                    + b2_ref[...], 0.0)
    h = jnp.maximum(jnp.dot(h, w3_ref[...], preferred_element_type=F32)
                    + b3_ref[...], 0.0)
    out_ref[...] = jnp.dot(h, w4_ref[...],
                           preferred_element_type=F32) + b4_ref[...]


def _edge_mlp(pre, ea, we, c0, w2, b2, w3, b3, w4, b4, blk):
    e, d = pre.shape
    de = ea.shape[1]
    grid = e // blk

    def wspec(shape):
        return pl.BlockSpec(shape, lambda i: (0, 0))

    return pl.pallas_call(
        _edge_body,
        grid=(grid,),
        in_specs=[
            pl.BlockSpec((blk, d), lambda i: (i, 0)),
            pl.BlockSpec((blk, de), lambda i: (i, 0)),
            wspec((de, d)), wspec((1, d)),
            wspec((d, d)), wspec((1, d)),
            wspec((d, d)), wspec((1, d)),
            wspec((d, d)), wspec((1, d)),
        ],
        out_specs=pl.BlockSpec((blk, d), lambda i: (i, 0)),
        out_shape=jax.ShapeDtypeStruct((e, d), F32),
    )(pre, ea, we, c0, w2, b2, w3, b3, w4, b4)


def _make_scatter(n_edges, n_nodes, d):
    steps = n_edges // (_NW * _CHUNK)
    epw = n_edges // _NW
    rows_per_sub = n_nodes // _NS
    mesh = plsc.VectorSubcoreMesh(core_axis_name="c", subcore_axis_name="s")

    @functools.partial(
        pl.kernel,
        mesh=mesh,
        out_type=jax.ShapeDtypeStruct((_NC, n_nodes, d), F32),
        scratch_types=[
            pltpu.VMEM((steps, _CHUNK), jnp.int32),
            pltpu.VMEM((_CHUNK, d), F32),
            pltpu.VMEM_SHARED((n_nodes, d), F32),
        ],
    )
    def scatter_k(eout_hbm, dst_hbm, zeros_hbm, out_hbm, didx, rbuf, acc):
        cid = lax.axis_index("c")
        sid = lax.axis_index("s")
        g = cid * _NS + sid
        zsl = pl.ds(sid * rows_per_sub, rows_per_sub)
        pltpu.sync_copy(zeros_hbm.at[zsl], acc.at[zsl])
        plsc.subcore_barrier()

        pltpu.sync_copy(dst_hbm.at[pl.ds(g * steps, steps)], didx)

        def step(j, carry):
            pltpu.sync_copy(eout_hbm.at[pl.ds(g * epw + j * _CHUNK, _CHUNK)],
                            rbuf)
            pltpu.sync_copy(rbuf, acc.at[didx.at[j]], add=True)
            return carry

        lax.fori_loop(0, steps, step, 0)
        plsc.subcore_barrier()
        pltpu.sync_copy(acc.at[zsl], out_hbm.at[cid, zsl])

    return scatter_k


def _node_body(x_ref, a0_ref, a1_ref, u_ref,
               wnx_ref, wna_ref, wnu_ref, bn1_ref,
               wn2_ref, bn2_ref, wn3_ref, bn3_ref, wn4_ref, bn4_ref,
               wgu_ref, wgn_ref, wge_ref, bg1_ref,
               wg2_ref, bg2_ref, wg3_ref, bg3_ref, wg4_ref, bg4_ref,
               inv_n_ref, inv_e_ref,
               nout_ref, uout_ref):
    agg = a0_ref[...] + a1_ref[...]
    u = u_ref[...]
    h = (jnp.dot(x_ref[...], wnx_ref[...], preferred_element_type=F32)
         + jnp.dot(agg, wna_ref[...], preferred_element_type=F32)
         + jnp.dot(u, wnu_ref[...], preferred_element_type=F32)
         + bn1_ref[...])
    h = jnp.maximum(h, 0.0)
    h = jnp.maximum(jnp.dot(h, wn2_ref[...], preferred_element_type=F32)
                    + bn2_ref[...], 0.0)
    h = jnp.maximum(jnp.dot(h, wn3_ref[...], preferred_element_type=F32)
                    + bn3_ref[...], 0.0)
    nout = jnp.dot(h, wn4_ref[...], preferred_element_type=F32) + bn4_ref[...]
    nout_ref[...] = nout

    mean_n = jnp.sum(nout, axis=0, keepdims=True) * inv_n_ref[0, 0]
    mean_e = jnp.sum(agg, axis=0, keepdims=True) * inv_e_ref[0, 0]
    g = (jnp.dot(u, wgu_ref[...], preferred_element_type=F32)
         + jnp.dot(mean_n, wgn_ref[...], preferred_element_type=F32)
         + jnp.dot(mean_e, wge_ref[...], preferred_element_type=F32)
         + bg1_ref[...])
    g = jnp.maximum(g, 0.0)
    g = jnp.maximum(jnp.dot(g, wg2_ref[...], preferred_element_type=F32)
                    + bg2_ref[...], 0.0)
    g = jnp.maximum(jnp.dot(g, wg3_ref[...], preferred_element_type=F32)
                    + bg3_ref[...], 0.0)
    uout_ref[...] = jnp.dot(g, wg4_ref[...],
                            preferred_element_type=F32) + bg4_ref[...]


def _node_global(x, a0, a1, u, node_w, glob_w, inv_n, inv_e):
    n, d = x.shape
    return pl.pallas_call(
        _node_body,
        out_shape=[
            jax.ShapeDtypeStruct((n, d), F32),
            jax.ShapeDtypeStruct((1, d), F32),
        ],
    )(x, a0, a1, u, *node_w, *glob_w, inv_n, inv_e)


def kernel(x, edge_attr, u, edge_params, node_params, global_params,
           edge_index):
    n_nodes, d_feat = x.shape
    n_edges, d_edge = edge_attr.shape
    d_u = u.shape[1]

    (w1, b1), (w2, b2), (w3, b3), (w4, b4) = edge_params
    d = w4.shape[1]
    we = w1[:d_edge]
    ws = w1[d_edge:d_edge + d_feat]
    wd = w1[d_edge + d_feat:d_edge + 2 * d_feat]
    wu = w1[d_edge + 2 * d_feat:]

    steps = n_edges // (_NW * _CHUNK)
    src = edge_index[0].reshape(_NW * steps, _CHUNK)
    dst = edge_index[1].reshape(_NW * steps, _CHUNK)

    xs, xd, c0 = _precompute(x, ws, wd, u, wu, b1.reshape(1, -1))

    pre = _make_gather(n_edges, d_feat)(xs, xd, src, dst)

    e_out = _edge_mlp(pre, edge_attr, we, c0,
                      w2, b2.reshape(1, -1), w3, b3.reshape(1, -1),
                      w4, b4.reshape(1, -1), blk=4000)

    zeros = jnp.zeros((n_nodes, d), F32)
    aggp = _make_scatter(n_edges, n_nodes, d)(e_out, dst, zeros)

    (nw1, nb1), (nw2, nb2), (nw3, nb3), (nw4, nb4) = node_params
    node_w = (nw1[:d_feat], nw1[d_feat:d_feat + d], nw1[d_feat + d:],
              nb1.reshape(1, -1),
              nw2, nb2.reshape(1, -1), nw3, nb3.reshape(1, -1),
              nw4, nb4.reshape(1, -1))
    (gw1, gb1), (gw2, gb2), (gw3, gb3), (gw4, gb4) = global_params
    glob_w = (gw1[:d_u], gw1[d_u:d_u + d], gw1[d_u + d:],
              gb1.reshape(1, -1),
              gw2, gb2.reshape(1, -1), gw3, gb3.reshape(1, -1),
              gw4, gb4.reshape(1, -1))

    inv_n = jnp.full((1, 1), 1.0 / n_nodes, F32)
    inv_e = jnp.full((1, 1), 1.0 / n_edges, F32)
    n_out, u_out = _node_global(x, aggp[0], aggp[1], u,
                                node_w, glob_w, inv_n, inv_e)

    return (e_out, n_out, u_out)


# speedup vs baseline: 3.5020x; 3.5020x over previous
import functools

import jax
import jax.numpy as jnp
from jax import lax
from jax.experimental import pallas as pl
from jax.experimental.pallas import tpu as pltpu
from jax.experimental.pallas import tpu_sc as plsc

F32 = jnp.float32

_NC = 2
_NS = 16
_NW = _NC * _NS

_CHUNK = 80


def _pre_body(x_ref, ws_ref, wd_ref, u_ref, wu_ref, b1_ref,
              xs_ref, xd_ref, c0_ref):
    x = x_ref[...]
    xs_ref[...] = jnp.dot(x, ws_ref[...], preferred_element_type=F32)
    xd_ref[...] = jnp.dot(x, wd_ref[...], preferred_element_type=F32)
    c0_ref[...] = jnp.dot(u_ref[...], wu_ref[...],
                          preferred_element_type=F32) + b1_ref[...]


def _precompute(x, ws, wd, u, wu, b1):
    n, d = x.shape
    return pl.pallas_call(
        _pre_body,
        out_shape=[
            jax.ShapeDtypeStruct((n, ws.shape[1]), F32),
            jax.ShapeDtypeStruct((n, wd.shape[1]), F32),
            jax.ShapeDtypeStruct((1, wu.shape[1]), F32),
        ],
    )(x, ws, wd, u, wu, b1)


def _make_gather(n_edges, d):
    steps = n_edges // (_NW * _CHUNK)
    epw = n_edges // _NW
    mesh = plsc.VectorSubcoreMesh(core_axis_name="c", subcore_axis_name="s")

    @functools.partial(
        pl.kernel,
        mesh=mesh,
        out_type=jax.ShapeDtypeStruct((n_edges, d), F32),
        scratch_types=[
            pltpu.VMEM((steps, _CHUNK), jnp.int32),
            pltpu.VMEM((steps, _CHUNK), jnp.int32),
            pltpu.VMEM((_CHUNK, d), F32),
            pltpu.VMEM((_CHUNK, d), F32),
            pltpu.SemaphoreType.DMA,
            pltpu.SemaphoreType.DMA,
        ],
    )
    def gather_k(xs_hbm, xd_hbm, src_hbm, dst_hbm, out_hbm,
                 sidx, didx, abuf, bbuf, sem_a, sem_b):
        wid = lax.axis_index("s") * _NC + lax.axis_index("c")
        pltpu.sync_copy(src_hbm.at[wid], sidx)
        pltpu.sync_copy(dst_hbm.at[wid], didx)

        def step(j, carry):
            ca = pltpu.async_copy(xs_hbm.at[sidx.at[j]], abuf, sem_a)
            cb = pltpu.async_copy(xd_hbm.at[didx.at[j]], bbuf, sem_b)
            ca.wait()
            cb.wait()

            def add_row(r, c2):
                for cc in range(d // 16):
                    sl = pl.ds(cc * 16, 16)
                    abuf[r, sl] = abuf[r, sl] + bbuf[r, sl]
                return c2

            lax.fori_loop(0, _CHUNK, add_row, 0)
            pltpu.sync_copy(abuf,
                            out_hbm.at[pl.ds(wid * epw + j * _CHUNK, _CHUNK)])
            return carry

        lax.fori_loop(0, steps, step, 0)

    return gather_k


def _edge_body(pre_ref, ea_ref, we_ref, c0_ref, w2_ref, b2_ref,
               w3_ref, b3_ref, w4_ref, b4_ref, out_ref):
    h = (pre_ref[...]
         + jnp.dot(ea_ref[...], we_ref[...], preferred_element_type=F32)
         + c0_ref[...])
    h = jnp.maximum(h, 0.0)
    h = jnp.maximum(jnp.dot(h, w2_ref[...], preferred_element_type=F32)
                    + b2_ref[...], 0.0)
    h = jnp.maximum(jnp.dot(h, w3_ref[...], preferred_element_type=F32)
                    + b3_ref[...], 0.0)
    out_ref[...] = jnp.dot(h, w4_ref[...],
                           preferred_element_type=F32) + b4_ref[...]


def _edge_mlp(pre, ea, we, c0, w2, b2, w3, b3, w4, b4, blk):
    e, d = pre.shape
    de = ea.shape[1]
    grid = e // blk

    def wspec(shape):
        return pl.BlockSpec(shape, lambda i: (0, 0))

    return pl.pallas_call(
        _edge_body,
        grid=(grid,),
        in_specs=[
            pl.BlockSpec((blk, d), lambda i: (i, 0)),
            pl.BlockSpec((blk, de), lambda i: (i, 0)),
            wspec((de, d)), wspec((1, d)),
            wspec((d, d)), wspec((1, d)),
            wspec((d, d)), wspec((1, d)),
            wspec((d, d)), wspec((1, d)),
        ],
        out_specs=pl.BlockSpec((blk, d), lambda i: (i, 0)),
        out_shape=jax.ShapeDtypeStruct((e, d), F32),
    )(pre, ea, we, c0, w2, b2, w3, b3, w4, b4)


def _make_scatter(n_edges, n_nodes, d):
    steps = n_edges // (_NW * _CHUNK)
    epw = n_edges // _NW
    rps = (n_nodes // (8 * _NS)) * 8
    rem = n_nodes - rps * _NS
    mesh = plsc.VectorSubcoreMesh(core_axis_name="c", subcore_axis_name="s")

    @functools.partial(
        pl.kernel,
        mesh=mesh,
        out_type=jax.ShapeDtypeStruct((_NC, n_nodes, d), F32),
        scratch_types=[
            pltpu.VMEM((steps, _CHUNK), jnp.int32),
            pltpu.VMEM((_CHUNK, d), F32),
            pltpu.VMEM_SHARED((n_nodes, d), F32),
        ],
    )
    def scatter_k(eout_hbm, dst_hbm, zeros_hbm, out_hbm, didx, rbuf, acc):
        cid = lax.axis_index("c")
        sid = lax.axis_index("s")
        g = cid * _NS + sid
        zsl = pl.ds(sid * rps, rps)
        tsl = pl.ds(_NS * rps, rem)
        pltpu.sync_copy(zeros_hbm.at[zsl], acc.at[zsl])

        @pl.when(sid == _NS - 1)
        def _zero_tail():
            pltpu.sync_copy(zeros_hbm.at[tsl], acc.at[tsl])

        plsc.subcore_barrier()

        pltpu.sync_copy(dst_hbm.at[g], didx)

        def step(j, carry):
            pltpu.sync_copy(eout_hbm.at[pl.ds(g * epw + j * _CHUNK, _CHUNK)],
                            rbuf)
            pltpu.sync_copy(rbuf, acc.at[didx.at[j]], add=True)
            return carry

        lax.fori_loop(0, steps, step, 0)
        plsc.subcore_barrier()
        pltpu.sync_copy(acc.at[zsl], out_hbm.at[cid, zsl])

        @pl.when(sid == _NS - 1)
        def _write_tail():
            pltpu.sync_copy(acc.at[tsl], out_hbm.at[cid, tsl])

    return scatter_k


def _node_body(x_ref, a0_ref, a1_ref, u_ref,
               wnx_ref, wna_ref, wnu_ref, bn1_ref,
               wn2_ref, bn2_ref, wn3_ref, bn3_ref, wn4_ref, bn4_ref,
               wgu_ref, wgn_ref, wge_ref, bg1_ref,
               wg2_ref, bg2_ref, wg3_ref, bg3_ref, wg4_ref, bg4_ref,
               inv_n_ref, inv_e_ref,
               nout_ref, uout_ref):
    agg = a0_ref[...] + a1_ref[...]
    u = u_ref[...]
    h = (jnp.dot(x_ref[...], wnx_ref[...], preferred_element_type=F32)
         + jnp.dot(agg, wna_ref[...], preferred_element_type=F32)
         + jnp.dot(u, wnu_ref[...], preferred_element_type=F32)
         + bn1_ref[...])
    h = jnp.maximum(h, 0.0)
    h = jnp.maximum(jnp.dot(h, wn2_ref[...], preferred_element_type=F32)
                    + bn2_ref[...], 0.0)
    h = jnp.maximum(jnp.dot(h, wn3_ref[...], preferred_element_type=F32)
                    + bn3_ref[...], 0.0)
    nout = jnp.dot(h, wn4_ref[...], preferred_element_type=F32) + bn4_ref[...]
    nout_ref[...] = nout

    mean_n = jnp.sum(nout, axis=0, keepdims=True) * inv_n_ref[0, 0]
    mean_e = jnp.sum(agg, axis=0, keepdims=True) * inv_e_ref[0, 0]
    g = (jnp.dot(u, wgu_ref[...], preferred_element_type=F32)
         + jnp.dot(mean_n, wgn_ref[...], preferred_element_type=F32)
         + jnp.dot(mean_e, wge_ref[...], preferred_element_type=F32)
         + bg1_ref[...])
    g = jnp.maximum(g, 0.0)
    g = jnp.maximum(jnp.dot(g, wg2_ref[...], preferred_element_type=F32)
                    + bg2_ref[...], 0.0)
    g = jnp.maximum(jnp.dot(g, wg3_ref[...], preferred_element_type=F32)
                    + bg3_ref[...], 0.0)
    uout_ref[...] = jnp.dot(g, wg4_ref[...],
                            preferred_element_type=F32) + bg4_ref[...]


def _node_global(x, a0, a1, u, node_w, glob_w, inv_n, inv_e):
    n, d = x.shape
    return pl.pallas_call(
        _node_body,
        out_shape=[
            jax.ShapeDtypeStruct((n, d), F32),
            jax.ShapeDtypeStruct((1, d), F32),
        ],
    )(x, a0, a1, u, *node_w, *glob_w, inv_n, inv_e)


def kernel(x, edge_attr, u, edge_params, node_params, global_params,
           edge_index):
    n_nodes, d_feat = x.shape
    n_edges, d_edge = edge_attr.shape
    d_u = u.shape[1]

    (w1, b1), (w2, b2), (w3, b3), (w4, b4) = edge_params
    d = w4.shape[1]
    we = w1[:d_edge]
    ws = w1[d_edge:d_edge + d_feat]
    wd = w1[d_edge + d_feat:d_edge + 2 * d_feat]
    wu = w1[d_edge + 2 * d_feat:]

    steps = n_edges // (_NW * _CHUNK)
    src = edge_index[0].reshape(_NW, steps, _CHUNK)
    dst = edge_index[1].reshape(_NW, steps, _CHUNK)

    xs, xd, c0 = _precompute(x, ws, wd, u, wu, b1.reshape(1, -1))

    pre = _make_gather(n_edges, d_feat)(xs, xd, src, dst)

    e_out = _edge_mlp(pre, edge_attr, we, c0,
                      w2, b2.reshape(1, -1), w3, b3.reshape(1, -1),
                      w4, b4.reshape(1, -1), blk=4000)

    zeros = jnp.zeros((n_nodes, d), F32)
    aggp = _make_scatter(n_edges, n_nodes, d)(e_out, dst, zeros)

    (nw1, nb1), (nw2, nb2), (nw3, nb3), (nw4, nb4) = node_params
    node_w = (nw1[:d_feat], nw1[d_feat:d_feat + d], nw1[d_feat + d:],
              nb1.reshape(1, -1),
              nw2, nb2.reshape(1, -1), nw3, nb3.reshape(1, -1),
              nw4, nb4.reshape(1, -1))
    (gw1, gb1), (gw2, gb2), (gw3, gb3), (gw4, gb4) = global_params
    glob_w = (gw1[:d_u], gw1[d_u:d_u + d], gw1[d_u + d:],
              gb1.reshape(1, -1),
              gw2, gb2.reshape(1, -1), gw3, gb3.reshape(1, -1),
              gw4, gb4.reshape(1, -1))

    inv_n = jnp.full((1, 1), 1.0 / n_nodes, F32)
    inv_e = jnp.full((1, 1), 1.0 / n_edges, F32)
    n_out, u_out = _node_global(x, aggp[0], aggp[1], u,
                                node_w, glob_w, inv_n, inv_e)

    return (e_out, n_out, u_out)


# speedup vs baseline: 4.6383x; 1.3244x over previous
import functools

import jax
import jax.numpy as jnp
from jax import lax
from jax.experimental import pallas as pl
from jax.experimental.pallas import tpu as pltpu
from jax.experimental.pallas import tpu_sc as plsc

F32 = jnp.float32

_NC = 2
_NS = 16
_NW = _NC * _NS

_CHUNK = 80


def _pre_body(x_ref, ws_ref, wd_ref, u_ref, wu_ref, b1_ref,
              xs_ref, xd_ref, c0_ref):
    x = x_ref[...]
    xs_ref[...] = jnp.dot(x, ws_ref[...], preferred_element_type=F32)
    xd_ref[...] = jnp.dot(x, wd_ref[...], preferred_element_type=F32)
    c0_ref[...] = jnp.dot(u_ref[...], wu_ref[...],
                          preferred_element_type=F32) + b1_ref[...]


def _precompute(x, ws, wd, u, wu, b1):
    n, d = x.shape
    return pl.pallas_call(
        _pre_body,
        out_shape=[
            jax.ShapeDtypeStruct((n, ws.shape[1]), F32),
            jax.ShapeDtypeStruct((n, wd.shape[1]), F32),
            jax.ShapeDtypeStruct((1, wu.shape[1]), F32),
        ],
    )(x, ws, wd, u, wu, b1)


def _make_gather(n_edges, d):
    steps = n_edges // (_NW * _CHUNK)
    epw = n_edges // _NW
    mesh = plsc.VectorSubcoreMesh(core_axis_name="c", subcore_axis_name="s")

    @functools.partial(
        pl.kernel,
        mesh=mesh,
        out_type=jax.ShapeDtypeStruct((n_edges, d), F32),
        scratch_types=[
            pltpu.VMEM((steps, _CHUNK), jnp.int32),
            pltpu.VMEM((steps, _CHUNK), jnp.int32),
            pltpu.VMEM((2, _CHUNK, d), F32),
            pltpu.VMEM((2, _CHUNK, d), F32),
            pltpu.SemaphoreType.DMA((2,)),
            pltpu.SemaphoreType.DMA((2,)),
            pltpu.SemaphoreType.DMA((2,)),
        ],
    )
    def gather_k(xs_hbm, xd_hbm, src_hbm, dst_hbm, out_hbm,
                 sidx, didx, abuf, bbuf, sem_a, sem_b, sem_o):
        wid = lax.axis_index("s") * _NC + lax.axis_index("c")
        pltpu.sync_copy(src_hbm.at[wid], sidx)
        pltpu.sync_copy(dst_hbm.at[wid], didx)

        def issue(j, b):
            pltpu.async_copy(xs_hbm.at[sidx.at[j]], abuf.at[b], sem_a.at[b])
            pltpu.async_copy(xd_hbm.at[didx.at[j]], bbuf.at[b], sem_b.at[b])

        def out_slice(j):
            return out_hbm.at[pl.ds(wid * epw + j * _CHUNK, _CHUNK)]

        def wait_gather(j, b):
            pltpu.make_async_copy(xs_hbm.at[sidx.at[j]], abuf.at[b],
                                  sem_a.at[b]).wait()
            pltpu.make_async_copy(xd_hbm.at[didx.at[j]], bbuf.at[b],
                                  sem_b.at[b]).wait()

        def wait_out(j, b):
            pltpu.make_async_copy(abuf.at[b], out_slice(j),
                                  sem_o.at[b]).wait()

        def add_and_write(j, b):
            def add_row(r, c2):
                for cc in range(d // 16):
                    sl = pl.ds(cc * 16, 16)
                    abuf[b, r, sl] = abuf[b, r, sl] + bbuf[b, r, sl]
                return c2

            lax.fori_loop(0, _CHUNK, add_row, 0)
            pltpu.async_copy(abuf.at[b], out_slice(j), sem_o.at[b])

        issue(0, 0)

        def step2(t, carry):
            j0 = 2 * t

            @pl.when(t > 0)
            def _():
                wait_out(j0 - 1, 1)

            issue(j0 + 1, 1)
            wait_gather(j0, 0)
            add_and_write(j0, 0)

            wait_out(j0, 0)
            issue(j0 + 2, 0)
            wait_gather(j0 + 1, 1)
            add_and_write(j0 + 1, 1)
            return carry

        lax.fori_loop(0, steps // 2, step2, 0)
        wait_gather(steps - 1, 0)
        add_and_write(steps - 1, 0)
        wait_out(steps - 2, 1)
        wait_out(steps - 1, 0)

    return gather_k


def _edge_body(pre_ref, ea_ref, we_ref, c0_ref, w2_ref, b2_ref,
               w3_ref, b3_ref, w4_ref, b4_ref, out_ref):
    h = (pre_ref[...]
         + jnp.dot(ea_ref[...], we_ref[...], preferred_element_type=F32)
         + c0_ref[...])
    h = jnp.maximum(h, 0.0)
    h = jnp.maximum(jnp.dot(h, w2_ref[...], preferred_element_type=F32)
                    + b2_ref[...], 0.0)
    h = jnp.maximum(jnp.dot(h, w3_ref[...], preferred_element_type=F32)
                    + b3_ref[...], 0.0)
    out_ref[...] = jnp.dot(h, w4_ref[...],
                           preferred_element_type=F32) + b4_ref[...]


def _edge_mlp(pre, ea, we, c0, w2, b2, w3, b3, w4, b4, blk):
    e, d = pre.shape
    de = ea.shape[1]
    grid = e // blk

    def wspec(shape):
        return pl.BlockSpec(shape, lambda i: (0, 0))

    return pl.pallas_call(
        _edge_body,
        grid=(grid,),
        in_specs=[
            pl.BlockSpec((blk, d), lambda i: (i, 0)),
            pl.BlockSpec((blk, de), lambda i: (i, 0)),
            wspec((de, d)), wspec((1, d)),
            wspec((d, d)), wspec((1, d)),
            wspec((d, d)), wspec((1, d)),
            wspec((d, d)), wspec((1, d)),
        ],
        out_specs=pl.BlockSpec((blk, d), lambda i: (i, 0)),
        out_shape=jax.ShapeDtypeStruct((e, d), F32),
    )(pre, ea, we, c0, w2, b2, w3, b3, w4, b4)


def _make_scatter(n_edges, n_nodes, d):
    steps = n_edges // (_NW * _CHUNK)
    epw = n_edges // _NW
    rps = (n_nodes // (8 * _NS)) * 8
    rem = n_nodes - rps * _NS
    mesh = plsc.VectorSubcoreMesh(core_axis_name="c", subcore_axis_name="s")

    @functools.partial(
        pl.kernel,
        mesh=mesh,
        out_type=jax.ShapeDtypeStruct((_NC, n_nodes, d), F32),
        scratch_types=[
            pltpu.VMEM((steps, _CHUNK), jnp.int32),
            pltpu.VMEM((2, _CHUNK, d), F32),
            pltpu.VMEM_SHARED((n_nodes, d), F32),
            pltpu.SemaphoreType.DMA((2,)),
            pltpu.SemaphoreType.DMA((2,)),
        ],
    )
    def scatter_k(eout_hbm, dst_hbm, zeros_hbm, out_hbm,
                  didx, rbuf, acc, sem_r, sem_w):
        cid = lax.axis_index("c")
        sid = lax.axis_index("s")
        g = cid * _NS + sid
        zsl = pl.ds(sid * rps, rps)
        tsl = pl.ds(_NS * rps, rem)
        pltpu.sync_copy(zeros_hbm.at[zsl], acc.at[zsl])

        @pl.when(sid == _NS - 1)
        def _zero_tail():
            pltpu.sync_copy(zeros_hbm.at[tsl], acc.at[tsl])

        plsc.subcore_barrier()

        pltpu.sync_copy(dst_hbm.at[g], didx)

        def in_slice(j):
            return eout_hbm.at[pl.ds(g * epw + j * _CHUNK, _CHUNK)]

        def wait_read(j, b):
            pltpu.make_async_copy(in_slice(j), rbuf.at[b], sem_r.at[b]).wait()

        def wait_add(j, b):
            pltpu.make_async_copy(rbuf.at[b], acc.at[didx.at[j]],
                                  sem_w.at[b]).wait()

        def start_add(j, b):
            pltpu.async_copy(rbuf.at[b], acc.at[didx.at[j]], sem_w.at[b],
                             add=True)

        pltpu.async_copy(in_slice(0), rbuf.at[0], sem_r.at[0])

        def step2(t, carry):
            j0 = 2 * t

            @pl.when(t > 0)
            def _():
                wait_add(j0 - 1, 1)

            pltpu.async_copy(in_slice(j0 + 1), rbuf.at[1], sem_r.at[1])
            wait_read(j0, 0)
            start_add(j0, 0)

            wait_add(j0, 0)
            pltpu.async_copy(in_slice(j0 + 2), rbuf.at[0], sem_r.at[0])
            wait_read(j0 + 1, 1)
            start_add(j0 + 1, 1)
            return carry

        lax.fori_loop(0, steps // 2, step2, 0)
        wait_read(steps - 1, 0)
        start_add(steps - 1, 0)
        wait_add(steps - 2, 1)
        wait_add(steps - 1, 0)
        plsc.subcore_barrier()
        pltpu.sync_copy(acc.at[zsl], out_hbm.at[cid, zsl])

        @pl.when(sid == _NS - 1)
        def _write_tail():
            pltpu.sync_copy(acc.at[tsl], out_hbm.at[cid, tsl])

    return scatter_k


def _node_body(x_ref, a0_ref, a1_ref, u_ref,
               wnx_ref, wna_ref, wnu_ref, bn1_ref,
               wn2_ref, bn2_ref, wn3_ref, bn3_ref, wn4_ref, bn4_ref,
               wgu_ref, wgn_ref, wge_ref, bg1_ref,
               wg2_ref, bg2_ref, wg3_ref, bg3_ref, wg4_ref, bg4_ref,
               inv_n_ref, inv_e_ref,
               nout_ref, uout_ref):
    agg = a0_ref[...] + a1_ref[...]
    u = u_ref[...]
    h = (jnp.dot(x_ref[...], wnx_ref[...], preferred_element_type=F32)
         + jnp.dot(agg, wna_ref[...], preferred_element_type=F32)
         + jnp.dot(u, wnu_ref[...], preferred_element_type=F32)
         + bn1_ref[...])
    h = jnp.maximum(h, 0.0)
    h = jnp.maximum(jnp.dot(h, wn2_ref[...], preferred_element_type=F32)
                    + bn2_ref[...], 0.0)
    h = jnp.maximum(jnp.dot(h, wn3_ref[...], preferred_element_type=F32)
                    + bn3_ref[...], 0.0)
    nout = jnp.dot(h, wn4_ref[...], preferred_element_type=F32) + bn4_ref[...]
    nout_ref[...] = nout

    mean_n = jnp.sum(nout, axis=0, keepdims=True) * inv_n_ref[0, 0]
    mean_e = jnp.sum(agg, axis=0, keepdims=True) * inv_e_ref[0, 0]
    g = (jnp.dot(u, wgu_ref[...], preferred_element_type=F32)
         + jnp.dot(mean_n, wgn_ref[...], preferred_element_type=F32)
         + jnp.dot(mean_e, wge_ref[...], preferred_element_type=F32)
         + bg1_ref[...])
    g = jnp.maximum(g, 0.0)
    g = jnp.maximum(jnp.dot(g, wg2_ref[...], preferred_element_type=F32)
                    + bg2_ref[...], 0.0)
    g = jnp.maximum(jnp.dot(g, wg3_ref[...], preferred_element_type=F32)
                    + bg3_ref[...], 0.0)
    uout_ref[...] = jnp.dot(g, wg4_ref[...],
                            preferred_element_type=F32) + bg4_ref[...]


def _node_global(x, a0, a1, u, node_w, glob_w, inv_n, inv_e):
    n, d = x.shape
    return pl.pallas_call(
        _node_body,
        out_shape=[
            jax.ShapeDtypeStruct((n, d), F32),
            jax.ShapeDtypeStruct((1, d), F32),
        ],
    )(x, a0, a1, u, *node_w, *glob_w, inv_n, inv_e)


def kernel(x, edge_attr, u, edge_params, node_params, global_params,
           edge_index):
    n_nodes, d_feat = x.shape
    n_edges, d_edge = edge_attr.shape
    d_u = u.shape[1]

    (w1, b1), (w2, b2), (w3, b3), (w4, b4) = edge_params
    d = w4.shape[1]
    we = w1[:d_edge]
    ws = w1[d_edge:d_edge + d_feat]
    wd = w1[d_edge + d_feat:d_edge + 2 * d_feat]
    wu = w1[d_edge + 2 * d_feat:]

    steps = n_edges // (_NW * _CHUNK)
    src = edge_index[0].reshape(_NW, steps, _CHUNK)
    dst = edge_index[1].reshape(_NW, steps, _CHUNK)

    xs, xd, c0 = _precompute(x, ws, wd, u, wu, b1.reshape(1, -1))

    pre = _make_gather(n_edges, d_feat)(xs, xd, src, dst)

    e_out = _edge_mlp(pre, edge_attr, we, c0,
                      w2, b2.reshape(1, -1), w3, b3.reshape(1, -1),
                      w4, b4.reshape(1, -1), blk=4000)

    zeros = jnp.zeros((n_nodes, d), F32)
    aggp = _make_scatter(n_edges, n_nodes, d)(e_out, dst, zeros)

    (nw1, nb1), (nw2, nb2), (nw3, nb3), (nw4, nb4) = node_params
    node_w = (nw1[:d_feat], nw1[d_feat:d_feat + d], nw1[d_feat + d:],
              nb1.reshape(1, -1),
              nw2, nb2.reshape(1, -1), nw3, nb3.reshape(1, -1),
              nw4, nb4.reshape(1, -1))
    (gw1, gb1), (gw2, gb2), (gw3, gb3), (gw4, gb4) = global_params
    glob_w = (gw1[:d_u], gw1[d_u:d_u + d], gw1[d_u + d:],
              gb1.reshape(1, -1),
              gw2, gb2.reshape(1, -1), gw3, gb3.reshape(1, -1),
              gw4, gb4.reshape(1, -1))

    inv_n = jnp.full((1, 1), 1.0 / n_nodes, F32)
    inv_e = jnp.full((1, 1), 1.0 / n_edges, F32)
    n_out, u_out = _node_global(x, aggp[0], aggp[1], u,
                                node_w, glob_w, inv_n, inv_e)

    return (e_out, n_out, u_out)
